# baseline (speedup 1.0000x reference)
import functools

import jax
import jax.numpy as jnp
from jax.experimental import pallas as pl
from jax.experimental.pallas import tpu as pltpu

N0, N1, N2 = 2048, 6144, 4096
D = 128
K = 128
CB1 = 256
SB1 = N1 // CB1
NC1 = 23
CB2 = 128
SB2 = N2 // CB2

_B2_L0 = SB1
_COMB0 = SB1 + SB2
_B1_L1 = _COMB0 + 1
_B2_L1 = _B1_L1 + SB1
_COMB1 = _B2_L1 + SB2
_STEPS = _COMB1 + 1

_BF = jnp.bfloat16

_dot = functools.partial(
    jax.lax.dot_general,
    preferred_element_type=jnp.float32,
    precision=jax.lax.Precision.DEFAULT,
)


def _mm(a, b):
    return _dot(a, b, dimension_numbers=(((1,), (0,)), ((), ())))


def _tmm(a, b):
    return _dot(a, b, dimension_numbers=(((0,), (0,)), ((), ())))


def _prep_body(x0, x1, x2, u0, u1, u2, l0, l1, l2,
               wi0, wi1, wi2, ws0, ws1, ws2, w01, w10, w12, w21,
               xb0, xb1, xb2, ub0, ub1, ub2, s0, s1, s2,
               wib0, wib1, wib2, wsb0, wsb1, wsb2,
               wb01, wb10, wb12, wb21):
    ones = jnp.ones((1, D), jnp.float32)
    for lam, s in ((l0, s0), (l1, s1), (l2, s2)):
        t = _tmm(lam[...], ones)
        s[0] = jnp.exp(-t)
        s[1] = jnp.exp(-2.0 * t)
    for src, dst in ((x0, xb0), (x1, xb1), (x2, xb2),
                     (u0, ub0), (u1, ub1), (u2, ub2),
                     (wi0, wib0), (wi1, wib1), (wi2, wib2),
                     (ws0, wsb0), (ws1, wsb1), (ws2, wsb2),
                     (w01, wb01), (w10, wb10), (w12, wb12), (w21, wb21)):
        dst[...] = src[...].astype(_BF)


def _body(x0, x1, x2, u0, u1, u2, s0, s1, s2, b1m, b2m,
          wi0, bi0, wi1, bi1, wi2, bi2, ws0, ws1, ws2, w01, w10, w12, w21,
          y0, y1, y2,
          h0s, h1s, h2s, a0s, a1s, a2s, a3s, b1c):
    i = pl.program_id(0)

    @pl.when(i == 0)
    def _proj():
        h0s[...] = (_mm(x0[...], wi0[...]) + bi0[...]).astype(_BF)
        h1s[...] = (_mm(x1[...], wi1[...]) + bi1[...]).astype(_BF)
        h2s[...] = (_mm(x2[...], wi2[...]) + bi2[...]).astype(_BF)

    def _b1_compute(blk, off):
        a0s[...] += _mm(blk, h1s[pl.ds(off, CB1), :])
        a1s[pl.ds(off, CB1), :] = _tmm(blk, h0s[...]).astype(_BF)

    @pl.when(i < SB1)
    def _b1_l0():
        j = i
        off = pl.multiple_of(j * CB1, CB1)
        blk = b1m[...].astype(_BF)

        @pl.when(i == 0)
        def _z():
            a0s[...] = jnp.zeros_like(a0s)

        @pl.when(j < NC1)
        def _cache():
            b1c[j] = blk

        _b1_compute(blk, off)

    @pl.when((i >= _B1_L1) & (i < _B1_L1 + SB1))
    def _b1_l1():
        j = i - _B1_L1
        off = pl.multiple_of(j * CB1, CB1)

        @pl.when(i == _B1_L1)
        def _z():
            a0s[...] = jnp.zeros_like(a0s)

        @pl.when(j < NC1)
        def _cached():
            _b1_compute(b1c[j], off)

        @pl.when(j >= NC1)
        def _streamed():
            _b1_compute(b1m[...].astype(_BF), off)

    in_b2 = ((i >= _B2_L0) & (i < _COMB0)) | ((i >= _B2_L1) & (i < _COMB1))

    @pl.when(in_b2)
    def _b2():
        j = jnp.where(i < _COMB0, i - _B2_L0, i - _B2_L1)
        off = pl.multiple_of(j * CB2, CB2)
        blk = b2m[...].astype(_BF)

        @pl.when((i == _B2_L0) | (i == _B2_L1))
        def _z():
            a2s[...] = jnp.zeros_like(a2s)

        a2s[...] += _mm(blk, h2s[pl.ds(off, CB2), :])
        a3s[pl.ds(off, CB2), :] = _tmm(blk, h1s[...]).astype(_BF)

    def _spectral(u, hs, sc, ws, l):
        xt = _tmm(u[...], hs[...])
        g = (_mm(xt.astype(_BF), ws[l, 0])
             + _mm((sc[0] * xt).astype(_BF), ws[l, 1])
             + _mm((sc[1] * xt).astype(_BF), ws[l, 2]))
        return _mm(u[...], g.astype(_BF))

    def _combine(l, o0, o1, o2, out_dtype):
        r0 = (_spectral(u0, h0s, s0, ws0, l)
              + _mm(a0s[...].astype(_BF), w01[l]))
        r1 = (_spectral(u1, h1s, s1, ws1, l)
              + _mm(a1s[...], w10[l])
              + _mm(a2s[...].astype(_BF), w12[l]))
        r2 = (_spectral(u2, h2s, s2, ws2, l)
              + _mm(a3s[...], w21[l]))
        o0[...] = r0.astype(out_dtype)
        o1[...] = r1.astype(out_dtype)
        o2[...] = r2.astype(out_dtype)

    @pl.when(i == _COMB0)
    def _c0():
        _combine(0, h0s, h1s, h2s, _BF)

    @pl.when(i == _COMB1)
    def _c1():
        _combine(1, y0, y1, y2, jnp.float32)


def _full(shape):
    nd = len(shape)
    return pl.BlockSpec(shape, lambda i, _nd=nd: (0,) * _nd)


def _full0(shape):
    nd = len(shape)
    return pl.BlockSpec(shape, lambda _nd=nd: (0,) * _nd)


def _b1_idx(i):
    j = jnp.where(i < _COMB0, jnp.clip(i, 0, SB1 - 1),
                  jnp.clip(i - _B1_L1, NC1, SB1 - 1))
    return (0, j)


def _b2_idx(i):
    j = jnp.where(i < _COMB0, i - _B2_L0, i - _B2_L1)
    return (0, jnp.clip(j, 0, SB2 - 1))


def kernel(x_0, x_1, x_2, lam_0, U_0, lam_1, U_1, lam_2, U_2, B1, B2,
           W_in_0, b_in_0, W_in_1, b_in_1, W_in_2, b_in_2,
           Ws0, Ws1, Ws2, W01, W10, W12, W21):
    f32 = jnp.float32
    prep_out_shape = [
        jax.ShapeDtypeStruct((N0, D), _BF), jax.ShapeDtypeStruct((N1, D), _BF),
        jax.ShapeDtypeStruct((N2, D), _BF),
        jax.ShapeDtypeStruct((N0, K), _BF), jax.ShapeDtypeStruct((N1, K), _BF),
        jax.ShapeDtypeStruct((N2, K), _BF),
        jax.ShapeDtypeStruct((2, K, D), f32), jax.ShapeDtypeStruct((2, K, D), f32),
        jax.ShapeDtypeStruct((2, K, D), f32),
        jax.ShapeDtypeStruct((D, D), _BF), jax.ShapeDtypeStruct((D, D), _BF),
        jax.ShapeDtypeStruct((D, D), _BF),
        jax.ShapeDtypeStruct((2, 3, D, D), _BF), jax.ShapeDtypeStruct((2, 3, D, D), _BF),
        jax.ShapeDtypeStruct((2, 3, D, D), _BF),
        jax.ShapeDtypeStruct((2, D, D), _BF), jax.ShapeDtypeStruct((2, D, D), _BF),
        jax.ShapeDtypeStruct((2, D, D), _BF), jax.ShapeDtypeStruct((2, D, D), _BF),
    ]
    prep = pl.pallas_call(
        _prep_body,
        out_shape=prep_out_shape,
    )(x_0, x_1, x_2, U_0, U_1, U_2,
      lam_0.reshape(1, K), lam_1.reshape(1, K), lam_2.reshape(1, K),
      W_in_0, W_in_1, W_in_2, Ws0, Ws1, Ws2, W01, W10, W12, W21)
    (xb0, xb1, xb2, ub0, ub1, ub2, s0, s1, s2,
     wib0, wib1, wib2, wsb0, wsb1, wsb2, wb01, wb10, wb12, wb21) = prep

    bi0 = b_in_0.reshape(1, D)
    bi1 = b_in_1.reshape(1, D)
    bi2 = b_in_2.reshape(1, D)

    in_specs = [
        _full((N0, D)), _full((N1, D)), _full((N2, D)),
        _full((N0, K)), _full((N1, K)), _full((N2, K)),
        _full((2, K, D)), _full((2, K, D)), _full((2, K, D)),
        pl.BlockSpec((N0, CB1), _b1_idx),
        pl.BlockSpec((N1, CB2), _b2_idx),
        _full((D, D)), _full((1, D)),
        _full((D, D)), _full((1, D)),
        _full((D, D)), _full((1, D)),
        _full((2, 3, D, D)), _full((2, 3, D, D)), _full((2, 3, D, D)),
        _full((2, D, D)), _full((2, D, D)), _full((2, D, D)), _full((2, D, D)),
    ]
    out_specs = [_full((N0, D)), _full((N1, D)), _full((N2, D))]
    out_shape = [
        jax.ShapeDtypeStruct((N0, D), jnp.float32),
        jax.ShapeDtypeStruct((N1, D), jnp.float32),
        jax.ShapeDtypeStruct((N2, D), jnp.float32),
    ]
    scratch_shapes = [
        pltpu.VMEM((N0, D), _BF),
        pltpu.VMEM((N1, D), _BF),
        pltpu.VMEM((N2, D), _BF),
        pltpu.VMEM((N0, D), jnp.float32),
        pltpu.VMEM((N1, D), _BF),
        pltpu.VMEM((N1, D), jnp.float32),
        pltpu.VMEM((N2, D), _BF),
        pltpu.VMEM((NC1, N0, CB1), _BF),
    ]

    y0, y1, y2 = pl.pallas_call(
        _body,
        grid=(_STEPS,),
        in_specs=in_specs,
        out_specs=out_specs,
        out_shape=out_shape,
        scratch_shapes=scratch_shapes,
        compiler_params=pltpu.CompilerParams(
            dimension_semantics=("arbitrary",),
        ),
    )(xb0, xb1, xb2, ub0, ub1, ub2, s0, s1, s2, B1, B2,
      wib0, bi0, wib1, bi1, wib2, bi2,
      wsb0, wsb1, wsb2, wb01, wb10, wb12, wb21)
    return (y0, y1, y2)


# speedup vs baseline: 1.1945x; 1.1945x over previous
import functools

import jax
import jax.numpy as jnp
from jax.experimental import pallas as pl
from jax.experimental.pallas import tpu as pltpu

N0, N1, N2 = 2048, 6144, 4096
D = 128
K = 128
CB1 = 512
SB1 = N1 // CB1
NC1 = 4
CB2 = 256
SB2 = N2 // CB2

_B2_L0 = SB1
_COMB0 = SB1 + SB2
_B1_L1 = _COMB0 + 1
_B2_L1 = _B1_L1 + SB1
_COMB1 = _B2_L1 + SB2
_STEPS = _COMB1 + 1

_BF = jnp.bfloat16

_dot = functools.partial(
    jax.lax.dot_general,
    preferred_element_type=jnp.float32,
    precision=jax.lax.Precision.DEFAULT,
)


def _mm(a, b):
    return _dot(a, b, dimension_numbers=(((1,), (0,)), ((), ())))


def _tmm(a, b):
    return _dot(a, b, dimension_numbers=(((0,), (0,)), ((), ())))


def _prep_body(x0, x1, x2, u0, u1, u2, l0, l1, l2,
               wi0, wi1, wi2, ws0, ws1, ws2, w01, w10, w12, w21,
               xb0, xb1, xb2, ub0, ub1, ub2, s0, s1, s2,
               wib0, wib1, wib2, wsb0, wsb1, wsb2,
               wb01, wb10, wb12, wb21):
    ones = jnp.ones((1, D), jnp.float32)
    for lam, s in ((l0, s0), (l1, s1), (l2, s2)):
        t = _tmm(lam[...], ones)
        s[0] = jnp.exp(-t)
        s[1] = jnp.exp(-2.0 * t)
    for src, dst in ((x0, xb0), (x1, xb1), (x2, xb2),
                     (u0, ub0), (u1, ub1), (u2, ub2),
                     (wi0, wib0), (wi1, wib1), (wi2, wib2),
                     (ws0, wsb0), (ws1, wsb1), (ws2, wsb2),
                     (w01, wb01), (w10, wb10), (w12, wb12), (w21, wb21)):
        dst[...] = src[...].astype(_BF)


def _body(x0, x1, x2, u0, u1, u2, s0, s1, s2, b1m, b2m,
          wi0, bi0, wi1, bi1, wi2, bi2, ws0, ws1, ws2, w01, w10, w12, w21,
          y0, y1, y2,
          h0s, h1s, h2s, a0s, a1s, a2s, a3s, b1c):
    i = pl.program_id(0)

    @pl.when(i == 0)
    def _proj():
        h0s[...] = (_mm(x0[...], wi0[...]) + bi0[...]).astype(_BF)
        h1s[...] = (_mm(x1[...], wi1[...]) + bi1[...]).astype(_BF)
        h2s[...] = (_mm(x2[...], wi2[...]) + bi2[...]).astype(_BF)

    def _b1_compute(blk, off):
        a0s[...] += _mm(blk, h1s[pl.ds(off, CB1), :])
        a1s[pl.ds(off, CB1), :] = _tmm(blk, h0s[...]).astype(_BF)

    @pl.when(i < SB1)
    def _b1_l0():
        j = i
        off = pl.multiple_of(j * CB1, CB1)
        blk = b1m[...].astype(_BF)

        @pl.when(i == 0)
        def _z():
            a0s[...] = jnp.zeros_like(a0s)

        @pl.when(j < NC1)
        def _cache():
            b1c[j] = blk

        _b1_compute(blk, off)

    @pl.when((i >= _B1_L1) & (i < _B1_L1 + SB1))
    def _b1_l1():
        j = i - _B1_L1
        off = pl.multiple_of(j * CB1, CB1)

        @pl.when(i == _B1_L1)
        def _z():
            a0s[...] = jnp.zeros_like(a0s)

        @pl.when(j < NC1)
        def _cached():
            _b1_compute(b1c[j], off)

        @pl.when(j >= NC1)
        def _streamed():
            _b1_compute(b1m[...].astype(_BF), off)

    in_b2 = ((i >= _B2_L0) & (i < _COMB0)) | ((i >= _B2_L1) & (i < _COMB1))

    @pl.when(in_b2)
    def _b2():
        j = jnp.where(i < _COMB0, i - _B2_L0, i - _B2_L1)
        off = pl.multiple_of(j * CB2, CB2)
        blk = b2m[...].astype(_BF)

        @pl.when((i == _B2_L0) | (i == _B2_L1))
        def _z():
            a2s[...] = jnp.zeros_like(a2s)

        a2s[...] += _mm(blk, h2s[pl.ds(off, CB2), :])
        a3s[pl.ds(off, CB2), :] = _tmm(blk, h1s[...]).astype(_BF)

    def _spectral(u, hs, sc, ws, l):
        xt = _tmm(u[...], hs[...])
        g = (_mm(xt.astype(_BF), ws[l, 0])
             + _mm((sc[0] * xt).astype(_BF), ws[l, 1])
             + _mm((sc[1] * xt).astype(_BF), ws[l, 2]))
        return _mm(u[...], g.astype(_BF))

    def _combine(l, o0, o1, o2, out_dtype):
        r0 = (_spectral(u0, h0s, s0, ws0, l)
              + _mm(a0s[...].astype(_BF), w01[l]))
        r1 = (_spectral(u1, h1s, s1, ws1, l)
              + _mm(a1s[...], w10[l])
              + _mm(a2s[...].astype(_BF), w12[l]))
        r2 = (_spectral(u2, h2s, s2, ws2, l)
              + _mm(a3s[...], w21[l]))
        o0[...] = r0.astype(out_dtype)
        o1[...] = r1.astype(out_dtype)
        o2[...] = r2.astype(out_dtype)

    @pl.when(i == _COMB0)
    def _c0():
        _combine(0, h0s, h1s, h2s, _BF)

    @pl.when(i == _COMB1)
    def _c1():
        _combine(1, y0, y1, y2, jnp.float32)


def _full(shape):
    nd = len(shape)
    return pl.BlockSpec(shape, lambda i, _nd=nd: (0,) * _nd)


def _full0(shape):
    nd = len(shape)
    return pl.BlockSpec(shape, lambda _nd=nd: (0,) * _nd)


def _b1_idx(i):
    j = jnp.where(i < _COMB0, jnp.clip(i, 0, SB1 - 1),
                  jnp.clip(i - _B1_L1, NC1, SB1 - 1))
    return (0, j)


def _b2_idx(i):
    j = jnp.where(i < _COMB0, i - _B2_L0, i - _B2_L1)
    return (0, jnp.clip(j, 0, SB2 - 1))


def kernel(x_0, x_1, x_2, lam_0, U_0, lam_1, U_1, lam_2, U_2, B1, B2,
           W_in_0, b_in_0, W_in_1, b_in_1, W_in_2, b_in_2,
           Ws0, Ws1, Ws2, W01, W10, W12, W21):
    f32 = jnp.float32
    prep_out_shape = [
        jax.ShapeDtypeStruct((N0, D), _BF), jax.ShapeDtypeStruct((N1, D), _BF),
        jax.ShapeDtypeStruct((N2, D), _BF),
        jax.ShapeDtypeStruct((N0, K), _BF), jax.ShapeDtypeStruct((N1, K), _BF),
        jax.ShapeDtypeStruct((N2, K), _BF),
        jax.ShapeDtypeStruct((2, K, D), f32), jax.ShapeDtypeStruct((2, K, D), f32),
        jax.ShapeDtypeStruct((2, K, D), f32),
        jax.ShapeDtypeStruct((D, D), _BF), jax.ShapeDtypeStruct((D, D), _BF),
        jax.ShapeDtypeStruct((D, D), _BF),
        jax.ShapeDtypeStruct((2, 3, D, D), _BF), jax.ShapeDtypeStruct((2, 3, D, D), _BF),
        jax.ShapeDtypeStruct((2, 3, D, D), _BF),
        jax.ShapeDtypeStruct((2, D, D), _BF), jax.ShapeDtypeStruct((2, D, D), _BF),
        jax.ShapeDtypeStruct((2, D, D), _BF), jax.ShapeDtypeStruct((2, D, D), _BF),
    ]
    prep = pl.pallas_call(
        _prep_body,
        out_shape=prep_out_shape,
    )(x_0, x_1, x_2, U_0, U_1, U_2,
      lam_0.reshape(1, K), lam_1.reshape(1, K), lam_2.reshape(1, K),
      W_in_0, W_in_1, W_in_2, Ws0, Ws1, Ws2, W01, W10, W12, W21)
    (xb0, xb1, xb2, ub0, ub1, ub2, s0, s1, s2,
     wib0, wib1, wib2, wsb0, wsb1, wsb2, wb01, wb10, wb12, wb21) = prep

    bi0 = b_in_0.reshape(1, D)
    bi1 = b_in_1.reshape(1, D)
    bi2 = b_in_2.reshape(1, D)

    in_specs = [
        _full((N0, D)), _full((N1, D)), _full((N2, D)),
        _full((N0, K)), _full((N1, K)), _full((N2, K)),
        _full((2, K, D)), _full((2, K, D)), _full((2, K, D)),
        pl.BlockSpec((N0, CB1), _b1_idx),
        pl.BlockSpec((N1, CB2), _b2_idx),
        _full((D, D)), _full((1, D)),
        _full((D, D)), _full((1, D)),
        _full((D, D)), _full((1, D)),
        _full((2, 3, D, D)), _full((2, 3, D, D)), _full((2, 3, D, D)),
        _full((2, D, D)), _full((2, D, D)), _full((2, D, D)), _full((2, D, D)),
    ]
    out_specs = [_full((N0, D)), _full((N1, D)), _full((N2, D))]
    out_shape = [
        jax.ShapeDtypeStruct((N0, D), jnp.float32),
        jax.ShapeDtypeStruct((N1, D), jnp.float32),
        jax.ShapeDtypeStruct((N2, D), jnp.float32),
    ]
    scratch_shapes = [
        pltpu.VMEM((N0, D), _BF),
        pltpu.VMEM((N1, D), _BF),
        pltpu.VMEM((N2, D), _BF),
        pltpu.VMEM((N0, D), jnp.float32),
        pltpu.VMEM((N1, D), _BF),
        pltpu.VMEM((N1, D), jnp.float32),
        pltpu.VMEM((N2, D), _BF),
        pltpu.VMEM((NC1, N0, CB1), _BF),
    ]

    y0, y1, y2 = pl.pallas_call(
        _body,
        grid=(_STEPS,),
        in_specs=in_specs,
        out_specs=out_specs,
        out_shape=out_shape,
        scratch_shapes=scratch_shapes,
        compiler_params=pltpu.CompilerParams(
            dimension_semantics=("arbitrary",),
        ),
    )(xb0, xb1, xb2, ub0, ub1, ub2, s0, s1, s2, B1, B2,
      wib0, bi0, wib1, bi1, wib2, bi2,
      wsb0, wsb1, wsb2, wb01, wb10, wb12, wb21)
    return (y0, y1, y2)


# speedup vs baseline: 1.2938x; 1.0832x over previous
import functools

import jax
import jax.numpy as jnp
from jax.experimental import pallas as pl
from jax.experimental.pallas import tpu as pltpu

N0, N1, N2 = 2048, 6144, 4096
NX = N0 + N1 + N2
D = 128
K = 128
CB1 = 512
SB1 = N1 // CB1
NC1 = 4
CB2 = 256
SB2 = N2 // CB2

_B2_L0 = SB1
_COMB0 = SB1 + SB2
_B1_L1 = _COMB0 + 1
_B2_L1 = _B1_L1 + SB1
_COMB1 = _B2_L1 + SB2
_STEPS = _COMB1 + 1

_WI = (0, D, 2 * D)
_WS = (3 * D, 9 * D, 15 * D)
_W01 = 21 * D
_W10 = 23 * D
_W12 = 25 * D
_W21 = 27 * D
_WROWS = 29 * D
_SC = (0, 2 * K, 4 * K)
_BI = 6 * K
_SROWS = 6 * K + 24

_BF = jnp.bfloat16

_dot = functools.partial(
    jax.lax.dot_general,
    preferred_element_type=jnp.float32,
    precision=jax.lax.Precision.DEFAULT,
)


def _mm(a, b):
    return _dot(a, b, dimension_numbers=(((1,), (0,)), ((), ())))


def _tmm(a, b):
    return _dot(a, b, dimension_numbers=(((0,), (0,)), ((), ())))


def _prep_body(x0, x1, x2, u0, u1, u2, l0, l1, l2,
               wi0, wi1, wi2, ws0, ws1, ws2, w01, w10, w12, w21,
               b0, b1, b2,
               xp, up, sp, wp):
    ones = jnp.ones((1, D), jnp.float32)
    for k, lam in enumerate((l0, l1, l2)):
        t = _tmm(lam[...], ones)
        sp[pl.ds(_SC[k], K), :] = jnp.exp(-t)
        sp[pl.ds(_SC[k] + K, K), :] = jnp.exp(-2.0 * t)
    for k, b in enumerate((b0, b1, b2)):
        sp[pl.ds(_BI + 8 * k, 1), :] = b[...]
    for off, n, src in ((0, N0, x0), (N0, N1, x1), (N0 + N1, N2, x2)):
        xp[pl.ds(off, n), :] = src[...].astype(_BF)
    for off, n, src in ((0, N0, u0), (N0, N1, u1), (N0 + N1, N2, u2)):
        up[pl.ds(off, n), :] = src[...].astype(_BF)
    for k, wi in enumerate((wi0, wi1, wi2)):
        wp[pl.ds(_WI[k], D), :] = wi[...].astype(_BF)
    for k, ws in enumerate((ws0, ws1, ws2)):
        wp[pl.ds(_WS[k], 6 * D), :] = ws[...].astype(_BF).reshape(6 * D, D)
    for off, w in ((_W01, w01), (_W10, w10), (_W12, w12), (_W21, w21)):
        wp[pl.ds(off, 2 * D), :] = w[...].astype(_BF).reshape(2 * D, D)


def _body(xp, up, sp, wp, b1m, b2m,
          y0, y1, y2,
          h0s, h1s, h2s, h0t, h1t, a0s, a1t, a2s, a3t, b1c):
    i = pl.program_id(0)

    @pl.when(i == 0)
    def _proj():
        for k, (off, n, hs) in enumerate(((0, N0, h0s), (N0, N1, h1s),
                                          (N0 + N1, N2, h2s))):
            h = (_mm(xp[pl.ds(off, n), :], wp[pl.ds(_WI[k], D), :])
                 + sp[pl.ds(_BI + 8 * k, 1), :]).astype(_BF)
            hs[...] = h
            if k == 0:
                h0t[...] = h.T
            elif k == 1:
                h1t[...] = h.T

    def _b1_compute(blk, off):
        a0s[...] += _mm(blk, h1s[pl.ds(off, CB1), :])
        a1t[:, pl.ds(off, CB1)] = _mm(h0t[...], blk).astype(_BF)

    @pl.when(i < SB1)
    def _b1_l0():
        j = i
        off = pl.multiple_of(j * CB1, CB1)
        blk = b1m[...].astype(_BF)

        @pl.when(i == 0)
        def _z():
            a0s[...] = jnp.zeros_like(a0s)

        @pl.when(j < NC1)
        def _cache():
            b1c[j] = blk

        _b1_compute(blk, off)

    @pl.when((i >= _B1_L1) & (i < _B1_L1 + SB1))
    def _b1_l1():
        j = i - _B1_L1
        off = pl.multiple_of(j * CB1, CB1)

        @pl.when(i == _B1_L1)
        def _z():
            a0s[...] = jnp.zeros_like(a0s)

        @pl.when(j < NC1)
        def _cached():
            _b1_compute(b1c[j], off)

        @pl.when(j >= NC1)
        def _streamed():
            _b1_compute(b1m[...].astype(_BF), off)

    in_b2 = ((i >= _B2_L0) & (i < _COMB0)) | ((i >= _B2_L1) & (i < _COMB1))

    @pl.when(in_b2)
    def _b2():
        j = jnp.where(i < _COMB0, i - _B2_L0, i - _B2_L1)
        off = pl.multiple_of(j * CB2, CB2)
        blk = b2m[...].astype(_BF)

        @pl.when((i == _B2_L0) | (i == _B2_L1))
        def _z():
            a2s[...] = jnp.zeros_like(a2s)

        a2s[...] += _mm(blk, h2s[pl.ds(off, CB2), :])
        a3t[:, pl.ds(off, CB2)] = _mm(h1t[...], blk).astype(_BF)

    def _spectral(uoff, n, hs, k, l):
        u = up[pl.ds(uoff, n), :]
        xt = _tmm(u, hs[...])
        def ws(r):
            return wp[pl.ds(_WS[k] + l * 3 * D + r * D, D), :]
        def sc(r):
            return sp[pl.ds(_SC[k] + (r - 1) * K, K), :]
        g = (_mm(xt.astype(_BF), ws(0))
             + _mm((sc(1) * xt).astype(_BF), ws(1))
             + _mm((sc(2) * xt).astype(_BF), ws(2)))
        return _mm(u, g.astype(_BF))

    def _combine(l, o0, o1, o2, out_dtype, update_t):
        def w(base):
            return wp[pl.ds(base + l * D, D), :]
        r0 = (_spectral(0, N0, h0s, 0, l)
              + _mm(a0s[...].astype(_BF), w(_W01)))
        r1 = (_spectral(N0, N1, h1s, 1, l)
              + _tmm(a1t[...], w(_W10))
              + _mm(a2s[...].astype(_BF), w(_W12)))
        r2 = (_spectral(N0 + N1, N2, h2s, 2, l)
              + _tmm(a3t[...], w(_W21)))
        o0[...] = r0.astype(out_dtype)
        o1[...] = r1.astype(out_dtype)
        o2[...] = r2.astype(out_dtype)
        if update_t:
            h0t[...] = r0.astype(_BF).T
            h1t[...] = r1.astype(_BF).T

    @pl.when(i == _COMB0)
    def _c0():
        _combine(0, h0s, h1s, h2s, _BF, True)

    @pl.when(i == _COMB1)
    def _c1():
        _combine(1, y0, y1, y2, jnp.float32, False)


def _full(shape):
    nd = len(shape)
    return pl.BlockSpec(shape, lambda i, _nd=nd: (0,) * _nd)


def _b1_idx(i):
    j = jnp.where(i < _COMB0, jnp.clip(i, 0, SB1 - 1),
                  jnp.clip(i - _B1_L1, NC1, SB1 - 1))
    return (0, j)


def _b2_idx(i):
    j = jnp.where(i < _COMB0, i - _B2_L0, i - _B2_L1)
    return (0, jnp.clip(j, 0, SB2 - 1))


def kernel(x_0, x_1, x_2, lam_0, U_0, lam_1, U_1, lam_2, U_2, B1, B2,
           W_in_0, b_in_0, W_in_1, b_in_1, W_in_2, b_in_2,
           Ws0, Ws1, Ws2, W01, W10, W12, W21):
    f32 = jnp.float32
    xp, up, sp, wp = pl.pallas_call(
        _prep_body,
        out_shape=[
            jax.ShapeDtypeStruct((NX, D), _BF),
            jax.ShapeDtypeStruct((NX, K), _BF),
            jax.ShapeDtypeStruct((_SROWS, D), f32),
            jax.ShapeDtypeStruct((_WROWS, D), _BF),
        ],
    )(x_0, x_1, x_2, U_0, U_1, U_2,
      lam_0.reshape(1, K), lam_1.reshape(1, K), lam_2.reshape(1, K),
      W_in_0, W_in_1, W_in_2, Ws0, Ws1, Ws2, W01, W10, W12, W21,
      b_in_0.reshape(1, D), b_in_1.reshape(1, D), b_in_2.reshape(1, D))

    y0, y1, y2 = pl.pallas_call(
        _body,
        grid=(_STEPS,),
        in_specs=[
            _full((NX, D)), _full((NX, K)),
            _full((_SROWS, D)), _full((_WROWS, D)),
            pl.BlockSpec((N0, CB1), _b1_idx),
            pl.BlockSpec((N1, CB2), _b2_idx),
        ],
        out_specs=[_full((N0, D)), _full((N1, D)), _full((N2, D))],
        out_shape=[
            jax.ShapeDtypeStruct((N0, D), f32),
            jax.ShapeDtypeStruct((N1, D), f32),
            jax.ShapeDtypeStruct((N2, D), f32),
        ],
        scratch_shapes=[
            pltpu.VMEM((N0, D), _BF),
            pltpu.VMEM((N1, D), _BF),
            pltpu.VMEM((N2, D), _BF),
            pltpu.VMEM((D, N0), _BF),
            pltpu.VMEM((D, N1), _BF),
            pltpu.VMEM((N0, D), f32),
            pltpu.VMEM((D, N1), _BF),
            pltpu.VMEM((N1, D), f32),
            pltpu.VMEM((D, N2), _BF),
            pltpu.VMEM((NC1, N0, CB1), _BF),
        ],
        compiler_params=pltpu.CompilerParams(
            dimension_semantics=("arbitrary",),
        ),
    )(xp, up, sp, wp, B1, B2)
    return (y0, y1, y2)


# speedup vs baseline: 1.4231x; 1.1000x over previous
import functools

import jax
import jax.numpy as jnp
from jax.experimental import pallas as pl
from jax.experimental.pallas import tpu as pltpu

N0, N1, N2 = 2048, 6144, 4096
NX = N0 + N1 + N2
D = 128
K = 128
CB1 = 512
SB1 = N1 // CB1
NC1 = 4
CB2 = 256
SB2 = N2 // CB2

_PH = max(SB1, SB2)
_COMB0 = _PH
_L1 = _COMB0 + 1
_COMB1 = _L1 + _PH
_STEPS = _COMB1 + 1

_WI = (0, D, 2 * D)
_WS = (3 * D, 9 * D, 15 * D)
_W01 = 21 * D
_W10 = 23 * D
_W12 = 25 * D
_W21 = 27 * D
_WROWS = 29 * D
_SC = (0, 2 * K, 4 * K)
_BI = 6 * K
_SROWS = 6 * K + 24

_BF = jnp.bfloat16

_dot = functools.partial(
    jax.lax.dot_general,
    preferred_element_type=jnp.float32,
    precision=jax.lax.Precision.DEFAULT,
)


def _mm(a, b):
    return _dot(a, b, dimension_numbers=(((1,), (0,)), ((), ())))


def _tmm(a, b):
    return _dot(a, b, dimension_numbers=(((0,), (0,)), ((), ())))


def _prep_body(x0, x1, x2, u0, u1, u2, l0, l1, l2,
               wi0, wi1, wi2, ws0, ws1, ws2, w01, w10, w12, w21,
               b0, b1, b2,
               xp, up, sp, wp):
    ones = jnp.ones((1, D), jnp.float32)
    for k, lam in enumerate((l0, l1, l2)):
        t = _tmm(lam[...], ones)
        sp[pl.ds(_SC[k], K), :] = jnp.exp(-t)
        sp[pl.ds(_SC[k] + K, K), :] = jnp.exp(-2.0 * t)
    for k, b in enumerate((b0, b1, b2)):
        sp[pl.ds(_BI + 8 * k, 1), :] = b[...]
    for off, n, src in ((0, N0, x0), (N0, N1, x1), (N0 + N1, N2, x2)):
        xp[pl.ds(off, n), :] = src[...].astype(_BF)
    for off, n, src in ((0, N0, u0), (N0, N1, u1), (N0 + N1, N2, u2)):
        up[pl.ds(off, n), :] = src[...].astype(_BF)
    for k, wi in enumerate((wi0, wi1, wi2)):
        wp[pl.ds(_WI[k], D), :] = wi[...].astype(_BF)
    for k, ws in enumerate((ws0, ws1, ws2)):
        wp[pl.ds(_WS[k], 6 * D), :] = ws[...].astype(_BF).reshape(6 * D, D)
    for off, w in ((_W01, w01), (_W10, w10), (_W12, w12), (_W21, w21)):
        wp[pl.ds(off, 2 * D), :] = w[...].astype(_BF).reshape(2 * D, D)


def _body(xp, up, sp, wp, b1m, b2m,
          y0, y1, y2,
          h0s, h1s, h2s, h0t, h1t, a0s, a1t, a2s, a3t, b1c):
    i = pl.program_id(0)

    @pl.when(i == 0)
    def _proj():
        for k, (off, n, hs) in enumerate(((0, N0, h0s), (N0, N1, h1s),
                                          (N0 + N1, N2, h2s))):
            h = (_mm(xp[pl.ds(off, n), :], wp[pl.ds(_WI[k], D), :])
                 + sp[pl.ds(_BI + 8 * k, 1), :]).astype(_BF)
            hs[...] = h
            if k == 0:
                h0t[...] = h.T
            elif k == 1:
                h1t[...] = h.T

    def _b1_compute(blk, off):
        a0s[...] += _mm(blk, h1s[pl.ds(off, CB1), :])
        a1t[:, pl.ds(off, CB1)] = _mm(h0t[...], blk).astype(_BF)

    @pl.when(i < SB1)
    def _b1_l0():
        j = i
        off = pl.multiple_of(j * CB1, CB1)
        blk = b1m[...].astype(_BF)

        @pl.when(i == 0)
        def _z():
            a0s[...] = jnp.zeros_like(a0s)

        @pl.when(j < NC1)
        def _cache():
            b1c[j] = blk

        _b1_compute(blk, off)

    @pl.when((i >= _L1) & (i < _L1 + SB1))
    def _b1_l1():
        j = i - _L1
        off = pl.multiple_of(j * CB1, CB1)

        @pl.when(i == _L1)
        def _z():
            a0s[...] = jnp.zeros_like(a0s)

        @pl.when(j < NC1)
        def _cached():
            _b1_compute(b1c[j], off)

        @pl.when(j >= NC1)
        def _streamed():
            _b1_compute(b1m[...].astype(_BF), off)

    in_b2 = ((i < SB2)) | ((i >= _L1) & (i < _L1 + SB2))

    @pl.when(in_b2)
    def _b2():
        j = jnp.where(i < _COMB0, i, i - _L1)
        off = pl.multiple_of(j * CB2, CB2)
        blk = b2m[...].astype(_BF)

        @pl.when((i == 0) | (i == _L1))
        def _z():
            a2s[...] = jnp.zeros_like(a2s)

        a2s[...] += _mm(blk, h2s[pl.ds(off, CB2), :])
        a3t[:, pl.ds(off, CB2)] = _mm(h1t[...], blk).astype(_BF)

    def _spectral(uoff, n, hs, k, l):
        u = up[pl.ds(uoff, n), :]
        xt = _tmm(u, hs[...])
        def ws(r):
            return wp[pl.ds(_WS[k] + l * 3 * D + r * D, D), :]
        def sc(r):
            return sp[pl.ds(_SC[k] + (r - 1) * K, K), :]
        g = (_mm(xt.astype(_BF), ws(0))
             + _mm((sc(1) * xt).astype(_BF), ws(1))
             + _mm((sc(2) * xt).astype(_BF), ws(2)))
        return _mm(u, g.astype(_BF))

    def _combine(l, o0, o1, o2, out_dtype, update_t):
        def w(base):
            return wp[pl.ds(base + l * D, D), :]
        r0 = (_spectral(0, N0, h0s, 0, l)
              + _mm(a0s[...].astype(_BF), w(_W01)))
        r1 = (_spectral(N0, N1, h1s, 1, l)
              + _tmm(a1t[...], w(_W10))
              + _mm(a2s[...].astype(_BF), w(_W12)))
        r2 = (_spectral(N0 + N1, N2, h2s, 2, l)
              + _tmm(a3t[...], w(_W21)))
        o0[...] = r0.astype(out_dtype)
        o1[...] = r1.astype(out_dtype)
        o2[...] = r2.astype(out_dtype)
        if update_t:
            h0t[...] = r0.astype(_BF).T
            h1t[...] = r1.astype(_BF).T

    @pl.when(i == _COMB0)
    def _c0():
        _combine(0, h0s, h1s, h2s, _BF, True)

    @pl.when(i == _COMB1)
    def _c1():
        _combine(1, y0, y1, y2, jnp.float32, False)


def _full(shape):
    nd = len(shape)
    return pl.BlockSpec(shape, lambda i, _nd=nd: (0,) * _nd)


def _b1_idx(i):
    j = jnp.where(i < _COMB0, jnp.clip(i, 0, SB1 - 1),
                  jnp.clip(i - _L1, NC1, SB1 - 1))
    return (0, j)


def _b2_idx(i):
    j = jnp.where(i < _COMB0, jnp.clip(i, 0, SB2 - 1),
                  jnp.clip(i - _L1, 0, SB2 - 1))
    return (0, j)


def kernel(x_0, x_1, x_2, lam_0, U_0, lam_1, U_1, lam_2, U_2, B1, B2,
           W_in_0, b_in_0, W_in_1, b_in_1, W_in_2, b_in_2,
           Ws0, Ws1, Ws2, W01, W10, W12, W21):
    f32 = jnp.float32
    xp, up, sp, wp = pl.pallas_call(
        _prep_body,
        out_shape=[
            jax.ShapeDtypeStruct((NX, D), _BF),
            jax.ShapeDtypeStruct((NX, K), _BF),
            jax.ShapeDtypeStruct((_SROWS, D), f32),
            jax.ShapeDtypeStruct((_WROWS, D), _BF),
        ],
    )(x_0, x_1, x_2, U_0, U_1, U_2,
      lam_0.reshape(1, K), lam_1.reshape(1, K), lam_2.reshape(1, K),
      W_in_0, W_in_1, W_in_2, Ws0, Ws1, Ws2, W01, W10, W12, W21,
      b_in_0.reshape(1, D), b_in_1.reshape(1, D), b_in_2.reshape(1, D))

    y0, y1, y2 = pl.pallas_call(
        _body,
        grid=(_STEPS,),
        in_specs=[
            _full((NX, D)), _full((NX, K)),
            _full((_SROWS, D)), _full((_WROWS, D)),
            pl.BlockSpec((N0, CB1), _b1_idx),
            pl.BlockSpec((N1, CB2), _b2_idx),
        ],
        out_specs=[_full((N0, D)), _full((N1, D)), _full((N2, D))],
        out_shape=[
            jax.ShapeDtypeStruct((N0, D), f32),
            jax.ShapeDtypeStruct((N1, D), f32),
            jax.ShapeDtypeStruct((N2, D), f32),
        ],
        scratch_shapes=[
            pltpu.VMEM((N0, D), _BF),
            pltpu.VMEM((N1, D), _BF),
            pltpu.VMEM((N2, D), _BF),
            pltpu.VMEM((D, N0), _BF),
            pltpu.VMEM((D, N1), _BF),
            pltpu.VMEM((N0, D), f32),
            pltpu.VMEM((D, N1), _BF),
            pltpu.VMEM((N1, D), f32),
            pltpu.VMEM((D, N2), _BF),
            pltpu.VMEM((NC1, N0, CB1), _BF),
        ],
        compiler_params=pltpu.CompilerParams(
            dimension_semantics=("arbitrary",),
        ),
    )(xp, up, sp, wp, B1, B2)
    return (y0, y1, y2)
